# baseline (speedup 1.0000x reference)
import jax
import jax.numpy as jnp
from jax.experimental import pallas as pl
from jax.experimental.pallas import tpu as pltpu

_BI = 512


def _prep_kernel(s1_ref, s2_ref, x_ref, y_ref):
    x = s1_ref[...]
    y = s2_ref[...]
    x2 = jnp.sum(x * x, axis=1, keepdims=True)
    y2 = jnp.sum(y * y, axis=1, keepdims=True)
    ones = jnp.ones_like(x2)
    x_ref[...] = jnp.concatenate([-2.0 * x, ones, x2], axis=1).astype(jnp.bfloat16)
    y_ref[...] = jnp.concatenate([y, y2, ones], axis=1).astype(jnp.bfloat16)


def _minblock_kernel(x_ref, y_ref, row_ref, col_ref):
    d2 = jax.lax.dot_general(
        x_ref[...], y_ref[...], (((1,), (1,)), ((), ())),
        preferred_element_type=jnp.float32,
        precision=jax.lax.Precision.DEFAULT,
    )
    row_ref[...] = jnp.min(d2, axis=1, keepdims=True).T
    col_ref[...] = jnp.min(d2, axis=0, keepdims=True)[None]


def _finalize_kernel(row_ref, col_ref, out_ref):
    r = jnp.sqrt(jnp.maximum(row_ref[...], 1e-12))
    c = jnp.sqrt(jnp.maximum(jnp.min(col_ref[...], axis=0), 1e-12))
    n = row_ref.shape[1]
    m = col_ref.shape[2]
    out_ref[...] = (jnp.sum(r) / n + jnp.sum(c) / m).reshape(1, 1)


@jax.jit
def kernel(set1, set2):
    s1 = set1.reshape(-1, set1.shape[-1])
    s2 = set2.reshape(-1, set2.shape[-1])
    n, dim = s1.shape
    m = s2.shape[0]
    d = dim + 2
    s1a, s2a = pl.pallas_call(
        _prep_kernel,
        out_shape=[
            jax.ShapeDtypeStruct((n, d), jnp.bfloat16),
            jax.ShapeDtypeStruct((m, d), jnp.bfloat16),
        ],
    )(s1, s2)
    ni = n // _BI
    row_min, col_partial = pl.pallas_call(
        _minblock_kernel,
        grid=(ni,),
        in_specs=[
            pl.BlockSpec((_BI, d), lambda i: (i, 0)),
            pl.BlockSpec((m, d), lambda i: (0, 0)),
        ],
        out_specs=[
            pl.BlockSpec((1, _BI), lambda i: (0, i)),
            pl.BlockSpec((1, 1, m), lambda i: (i, 0, 0)),
        ],
        out_shape=[
            jax.ShapeDtypeStruct((1, n), jnp.float32),
            jax.ShapeDtypeStruct((ni, 1, m), jnp.float32),
        ],
        compiler_params=pltpu.CompilerParams(
            dimension_semantics=("parallel",),
        ),
    )(s1a, s2a)
    out = pl.pallas_call(
        _finalize_kernel,
        out_shape=jax.ShapeDtypeStruct((1, 1), jnp.float32),
    )(row_min, col_partial)
    return out[0, 0]


# speedup vs baseline: 1.3141x; 1.3141x over previous
import jax
import jax.numpy as jnp
from jax.experimental import pallas as pl
from jax.experimental.pallas import tpu as pltpu

_BI = 1024


def _prep_kernel(s1_ref, s2_ref, x_ref, y_ref):
    x = s1_ref[...]
    y = s2_ref[...]
    x2 = jnp.sum(x * x, axis=1, keepdims=True)
    y2 = jnp.sum(y * y, axis=1, keepdims=True)
    x2h = x2.astype(jnp.float8_e4m3fn).astype(jnp.float32)
    x2l = x2 - x2h
    y2h = y2.astype(jnp.float8_e4m3fn).astype(jnp.float32)
    y2l = y2 - y2h
    ones = jnp.ones_like(x2)
    x_ref[...] = jnp.concatenate(
        [-2.0 * x, ones, ones, x2h, x2l], axis=1).astype(jnp.float8_e4m3fn)
    y_ref[...] = jnp.concatenate(
        [y, y2h, y2l, ones, ones], axis=1).astype(jnp.float8_e4m3fn)


def _minblock_kernel(x_ref, y_ref, row_ref, col_ref):
    d2 = jax.lax.dot_general(
        x_ref[...], y_ref[...], (((1,), (1,)), ((), ())),
        preferred_element_type=jnp.float32,
        precision=jax.lax.Precision.DEFAULT,
    )
    row_ref[...] = jnp.min(d2, axis=1, keepdims=True).T
    col_ref[...] = jnp.min(d2, axis=0, keepdims=True)[None]


def _finalize_kernel(row_ref, col_ref, out_ref):
    r = jnp.sqrt(jnp.maximum(row_ref[...], 1e-12))
    c = jnp.sqrt(jnp.maximum(jnp.min(col_ref[...], axis=0), 1e-12))
    n = row_ref.shape[1]
    m = col_ref.shape[2]
    out_ref[...] = (jnp.sum(r) / n + jnp.sum(c) / m).reshape(1, 1)


@jax.jit
def kernel(set1, set2):
    s1 = set1.reshape(-1, set1.shape[-1])
    s2 = set2.reshape(-1, set2.shape[-1])
    n, dim = s1.shape
    m = s2.shape[0]
    d = dim + 4
    s1a, s2a = pl.pallas_call(
        _prep_kernel,
        out_shape=[
            jax.ShapeDtypeStruct((n, d), jnp.float8_e4m3fn),
            jax.ShapeDtypeStruct((m, d), jnp.float8_e4m3fn),
        ],
    )(s1, s2)
    ni = n // _BI
    row_min, col_partial = pl.pallas_call(
        _minblock_kernel,
        grid=(ni,),
        in_specs=[
            pl.BlockSpec((_BI, d), lambda i: (i, 0)),
            pl.BlockSpec((m, d), lambda i: (0, 0)),
        ],
        out_specs=[
            pl.BlockSpec((1, _BI), lambda i: (0, i)),
            pl.BlockSpec((1, 1, m), lambda i: (i, 0, 0)),
        ],
        out_shape=[
            jax.ShapeDtypeStruct((1, n), jnp.float32),
            jax.ShapeDtypeStruct((ni, 1, m), jnp.float32),
        ],
        compiler_params=pltpu.CompilerParams(
            dimension_semantics=("parallel",),
        ),
    )(s1a, s2a)
    out = pl.pallas_call(
        _finalize_kernel,
        out_shape=jax.ShapeDtypeStruct((1, 1), jnp.float32),
    )(row_min, col_partial)
    return out[0, 0]


# speedup vs baseline: 1.3333x; 1.0146x over previous
import jax
import jax.numpy as jnp
from jax.experimental import pallas as pl
from jax.experimental.pallas import tpu as pltpu

_BI = 1024


def _prep_kernel(s1_ref, s2_ref, x_ref, y_ref):
    x = s1_ref[...]
    y = s2_ref[...]
    x2 = jnp.sum(x * x, axis=1, keepdims=True)
    y2 = jnp.sum(y * y, axis=1, keepdims=True)
    x2h = x2.astype(jnp.float8_e4m3fn).astype(jnp.float32)
    x2l = x2 - x2h
    y2h = y2.astype(jnp.float8_e4m3fn).astype(jnp.float32)
    y2l = y2 - y2h
    ones = jnp.ones_like(x2)
    x_ref[...] = jnp.concatenate(
        [-2.0 * x, ones, ones, x2h, x2l], axis=1).astype(jnp.float8_e4m3fn)
    y_ref[...] = jnp.concatenate(
        [y, y2h, y2l, ones, ones], axis=1).astype(jnp.float8_e4m3fn)


def _minblock_kernel(x_ref, y_ref, row_ref, col_ref):
    d2 = jax.lax.dot_general(
        x_ref[...], y_ref[...], (((1,), (1,)), ((), ())),
        preferred_element_type=jnp.float32,
        precision=jax.lax.Precision.DEFAULT,
    )
    row_ref[...] = jnp.min(d2, axis=1, keepdims=True).T
    col_ref[...] = jnp.min(d2, axis=0, keepdims=True)[None]


def _finalize_kernel(row_ref, col_ref, out_ref):
    r = jnp.sqrt(jnp.maximum(row_ref[...], 1e-12))
    c = jnp.sqrt(jnp.maximum(jnp.min(col_ref[...], axis=0), 1e-12))
    n = row_ref.shape[1]
    m = col_ref.shape[2]
    out_ref[...] = (jnp.sum(r) / n + jnp.sum(c) / m).reshape(1, 1)


@jax.jit
def kernel(set1, set2):
    s1 = set1.reshape(-1, set1.shape[-1])
    s2 = set2.reshape(-1, set2.shape[-1])
    n, dim = s1.shape
    m = s2.shape[0]
    d = dim + 4
    _PB = 2048
    s1a, s2a = pl.pallas_call(
        _prep_kernel,
        grid=(n // _PB,),
        in_specs=[
            pl.BlockSpec((_PB, dim), lambda i: (i, 0)),
            pl.BlockSpec((_PB, dim), lambda i: (i, 0)),
        ],
        out_specs=[
            pl.BlockSpec((_PB, d), lambda i: (i, 0)),
            pl.BlockSpec((_PB, d), lambda i: (i, 0)),
        ],
        out_shape=[
            jax.ShapeDtypeStruct((n, d), jnp.float8_e4m3fn),
            jax.ShapeDtypeStruct((m, d), jnp.float8_e4m3fn),
        ],
    )(s1, s2)
    ni = n // _BI
    row_min, col_partial = pl.pallas_call(
        _minblock_kernel,
        grid=(ni,),
        in_specs=[
            pl.BlockSpec((_BI, d), lambda i: (i, 0)),
            pl.BlockSpec((m, d), lambda i: (0, 0)),
        ],
        out_specs=[
            pl.BlockSpec((1, _BI), lambda i: (0, i)),
            pl.BlockSpec((1, 1, m), lambda i: (i, 0, 0)),
        ],
        out_shape=[
            jax.ShapeDtypeStruct((1, n), jnp.float32),
            jax.ShapeDtypeStruct((ni, 1, m), jnp.float32),
        ],
        compiler_params=pltpu.CompilerParams(
            dimension_semantics=("parallel",),
        ),
    )(s1a, s2a)
    out = pl.pallas_call(
        _finalize_kernel,
        out_shape=jax.ShapeDtypeStruct((1, 1), jnp.float32),
    )(row_min, col_partial)
    return out[0, 0]
